# Initial kernel scaffold; baseline (speedup 1.0000x reference)
import jax
import jax.numpy as jnp
from jax.experimental import pallas as pl


def kernel(x, W1, b1, W2, b2, W3, b3, Wg1, bg1, Wg2, bg2):
    raise NotImplementedError("write your pallas kernel here")


# speedup vs baseline: 1.1434x; 1.1434x over previous
import numpy as np

import jax
import jax.numpy as jnp
from jax.experimental import pallas as pl


def _pair_mats():
    s1 = np.zeros((16, 64), np.float32)
    s2 = np.zeros((16, 64), np.float32)
    s3 = np.zeros((64, 16), np.float32)
    d1 = np.zeros((16, 16), np.float32)
    d2 = np.zeros((16, 16), np.float32)
    for i in range(4):
        for j in range(4):
            p = 4 * i + j
            for d in range(4):
                c = 16 * i + 4 * j + d
                s1[4 * i + d, c] = 1.0
                s2[4 * j + d, c] = 1.0
                s3[c, p] = 1.0
            d1[5 * i, p] = 1.0
            d2[5 * j, p] = 1.0
    return s1, s2, s3, d1, d2


def _graph_kernel(x_ref, s1_ref, s2_ref, s3_ref, d1_ref, d2_ref, o_ref):
    step = pl.program_id(0)
    f32 = jnp.float32
    x = x_ref[...]
    y1 = jnp.dot(x, s1_ref[...], preferred_element_type=f32)
    y2 = jnp.dot(x, s2_ref[...], preferred_element_type=f32)
    dots = jnp.dot(y1 * y2, s3_ref[...], preferred_element_type=f32)
    n_i = jnp.dot(dots, d1_ref[...], preferred_element_type=f32)
    n_j = jnp.dot(dots, d2_ref[...], preferred_element_type=f32)
    denom = jnp.maximum(jnp.sqrt(n_i) * jnp.sqrt(n_j), 1e-8)
    ind = (dots / denom > 0.5).astype(f32)
    red = jnp.max(ind, axis=0, keepdims=True)

    @pl.when(step == 0)
    def _init():
        o_ref[...] = red

    @pl.when(step != 0)
    def _acc():
        o_ref[...] = jnp.maximum(o_ref[...], red)


def _main_kernel(x_ref, an_ref, w1_ref, b1_ref, w2_ref, b2_ref, w3_ref,
                 b3_ref, wg1_ref, bg1_ref, wg2_ref, bg2_ref,
                 emb_ref, out_ref):
    f32 = jnp.float32
    h1 = jnp.maximum(jnp.dot(x_ref[...], w1_ref[...],
                             preferred_element_type=f32) + b1_ref[...], 0.0)
    ts = []
    for i in range(4):
        hi = h1[:, 128 * i:128 * (i + 1)]
        h2 = jnp.maximum(jnp.dot(hi, w2_ref[...],
                                 preferred_element_type=f32) + b2_ref[...], 0.0)
        ei = jnp.dot(h2, w3_ref[...], preferred_element_type=f32) + b3_ref[...]
        emb_ref[:, 128 * i:128 * (i + 1)] = ei
        ts.append(jnp.dot(ei, wg1_ref[...], preferred_element_type=f32))
    ss = []
    for i in range(4):
        acc = an_ref[0, 4 * i] * ts[0]
        for j in range(1, 4):
            acc = acc + an_ref[0, 4 * i + j] * ts[j]
        g1 = jnp.maximum(acc + bg1_ref[...], 0.0)
        ss.append(jnp.dot(g1, wg2_ref[...], preferred_element_type=f32))
    outs = []
    for i in range(4):
        acc = an_ref[0, 4 * i] * ss[0]
        for j in range(1, 4):
            acc = acc + an_ref[0, 4 * i + j] * ss[j]
        outs.append(acc + bg2_ref[...])
    out_ref[...] = jnp.concatenate(outs, axis=1)


@jax.jit
def kernel(x, W1, b1, W2, b2, W3, b3, Wg1, bg1, Wg2, bg2):
    B = x.shape[0]
    x2 = x.reshape(B, 16)
    s1, s2, s3, d1, d2 = (jnp.asarray(m) for m in _pair_mats())
    bB1 = 8192
    part = pl.pallas_call(
        _graph_kernel,
        grid=(B // bB1,),
        in_specs=[
            pl.BlockSpec((bB1, 16), lambda i: (i, 0)),
            pl.BlockSpec((16, 64), lambda i: (0, 0)),
            pl.BlockSpec((16, 64), lambda i: (0, 0)),
            pl.BlockSpec((64, 16), lambda i: (0, 0)),
            pl.BlockSpec((16, 16), lambda i: (0, 0)),
            pl.BlockSpec((16, 16), lambda i: (0, 0)),
        ],
        out_specs=pl.BlockSpec((1, 16), lambda i: (0, 0)),
        out_shape=jax.ShapeDtypeStruct((1, 16), jnp.float32),
    )(x2, s1, s2, s3, d1, d2)

    E = part.reshape(4, 4) > 0.5
    off = ~jnp.eye(4, dtype=bool)
    A_hat = jnp.where(off, (E & off).astype(jnp.float32), 1.0)
    deg = jnp.sum(A_hat, axis=1)
    dinv = deg ** -0.5
    An = dinv[:, None] * A_hat * dinv[None, :]
    An16 = An.reshape(1, 16)

    W1b = jnp.kron(jnp.eye(4, dtype=jnp.float32), W1)
    b1t = jnp.tile(b1, 4).reshape(1, 512)

    bB = 2048
    const = lambda shape: pl.BlockSpec(shape, lambda i: tuple(0 for _ in shape))
    emb, out = pl.pallas_call(
        _main_kernel,
        grid=(B // bB,),
        in_specs=[
            pl.BlockSpec((bB, 16), lambda i: (i, 0)),
            const((1, 16)),
            const((16, 512)), const((1, 512)),
            const((128, 64)), const((1, 64)),
            const((64, 128)), const((1, 128)),
            const((128, 32)), const((1, 32)),
            const((32, 4)), const((1, 4)),
        ],
        out_specs=[
            pl.BlockSpec((bB, 512), lambda i: (i, 0)),
            pl.BlockSpec((bB, 16), lambda i: (i, 0)),
        ],
        out_shape=[
            jax.ShapeDtypeStruct((B, 512), jnp.float32),
            jax.ShapeDtypeStruct((B, 16), jnp.float32),
        ],
    )(x2, An16, W1b, b1t, W2, b2.reshape(1, -1), W3, b3.reshape(1, -1),
      Wg1, bg1.reshape(1, -1), Wg2, bg2.reshape(1, -1))

    return emb.reshape(B, 4, 128), out.reshape(B, 4, 4)
